# Initial kernel scaffold; baseline (speedup 1.0000x reference)
import jax
import jax.numpy as jnp
from jax.experimental import pallas as pl


def kernel(input_ids, table, W, b):
    raise NotImplementedError("write your pallas kernel here")


# speedup vs baseline: 13.2469x; 13.2469x over previous
import functools

import jax
import jax.numpy as jnp
from jax import lax
from jax.experimental import pallas as pl
from jax.experimental.pallas import tpu as pltpu
from jax.experimental.pallas import tpu_sc as plsc

_CPAD = 16


def _proj_body(tab_ref, w_ref, b_ref, out_ref):
    out_ref[...] = (
        jnp.dot(tab_ref[...], w_ref[...], preferred_element_type=jnp.float32)
        + b_ref[...]
    )


def _project_table(table, W, b, blk_rows):
    V, D = table.shape
    grid = (V + blk_rows - 1) // blk_rows
    w_pad = jnp.zeros((D, _CPAD), jnp.float32).at[:, : W.shape[1]].set(W)
    b_pad = jnp.zeros((1, _CPAD), jnp.float32).at[:, : W.shape[1]].set(b)
    return pl.pallas_call(
        _proj_body,
        grid=(grid,),
        in_specs=[
            pl.BlockSpec((blk_rows, D), lambda i: (i, 0)),
            pl.BlockSpec((D, _CPAD), lambda i: (0, 0)),
            pl.BlockSpec((1, _CPAD), lambda i: (0, 0)),
        ],
        out_specs=pl.BlockSpec((blk_rows, _CPAD), lambda i: (i, 0)),
        out_shape=jax.ShapeDtypeStruct((V, _CPAD), jnp.float32),
    )(table, w_pad, b_pad)


def _gather_rows(P, idx, n_classes, chunk):
    B = idx.shape[0]
    info = plsc.get_sparse_core_info()
    nw = info.num_cores * info.num_subcores
    bpw = B // nw
    nchunks = bpw // chunk
    mesh = plsc.VectorSubcoreMesh(core_axis_name="c", subcore_axis_name="s")

    @functools.partial(
        pl.kernel,
        mesh=mesh,
        out_type=jax.ShapeDtypeStruct((B * n_classes,), jnp.float32),
        compiler_params=pltpu.CompilerParams(
            use_tc_tiling_on_sc=False, needs_layout_passes=False
        ),
        scratch_types=[
            pltpu.VMEM((chunk,), jnp.int32),
            pltpu.VMEM((chunk, _CPAD), jnp.float32),
            pltpu.VMEM((chunk * n_classes,), jnp.float32),
            pltpu.SemaphoreType.DMA,
        ],
    )
    def k(p_hbm, idx_hbm, out_hbm, idx_v, rows_v, comp_v, sem):
        wid = lax.axis_index("s") * info.num_cores + lax.axis_index("c")
        base = wid * bpw
        lanes = jax.lax.iota(jnp.int32, 16)
        row_off = jax.lax.shift_right_logical(lanes, 1)
        col_idx = jax.lax.bitwise_and(lanes, 1)
        per_vec = 16 // n_classes

        def body(c, _):
            off = base + c * chunk
            pltpu.sync_copy(idx_hbm.at[pl.ds(off, chunk)], idx_v)
            pltpu.async_copy(p_hbm.at[idx_v], rows_v, sem).wait()

            def compact(t, _):
                row_idx = row_off + t * per_vec
                v = plsc.load_gather(rows_v, [row_idx, col_idx])
                comp_v[pl.ds(t * 16, 16)] = v
                return 0

            lax.fori_loop(0, chunk * n_classes // 16, compact, 0)
            pltpu.sync_copy(comp_v, out_hbm.at[pl.ds(off * n_classes, chunk * n_classes)])
            return 0

        lax.fori_loop(0, nchunks, body, 0)

    return k(P, idx)


def kernel(input_ids, table, W, b):
    batch, seq = input_ids.shape
    n_classes = W.shape[1]
    proj = _project_table(table, W, b, blk_rows=8000)
    idx = input_ids.reshape(-1).astype(jnp.int32)
    flat = _gather_rows(proj, idx, n_classes, chunk=6400)
    return flat.reshape(batch, seq, n_classes)
